# baseline (speedup 1.0000x reference)
import functools

import jax
import jax.numpy as jnp
from jax import lax
from jax.experimental import pallas as pl
from jax.experimental.pallas import tpu as pltpu
from jax.experimental.pallas import tpu_sc as plsc

_NEMB = 32
_BATCH = 16384
_NC = 2
_NS = 16
_LANES = 16
_NW = _NC * _NS
_BPW = _BATCH // _NW
_CHUNK = 128
_NCHUNK = _BPW // _CHUNK


@functools.partial(
    pl.kernel,
    out_type=jax.ShapeDtypeStruct((_BATCH,), jnp.float32),
    mesh=plsc.VectorSubcoreMesh(core_axis_name="c", subcore_axis_name="s"),
    compiler_params=pltpu.CompilerParams(
        needs_layout_passes=False, use_tc_tiling_on_sc=False),
    scratch_types=[
        pltpu.VMEM((_NCHUNK, _CHUNK), jnp.int32),
        pltpu.VMEM((_NCHUNK, _CHUNK), jnp.int32),
        pltpu.VMEM((_BPW, _NEMB), jnp.float32),
        pltpu.VMEM((_BPW, _NEMB), jnp.float32),
        pltpu.VMEM((_BPW,), jnp.float32),
        pltpu.SemaphoreType.DMA,
    ],
)
def _sc_embed_dot(champ1_hbm, champ2_hbm, w_hbm, out_hbm,
                  idx1_v, idx2_v, rows1_v, rows2_v, out_v, sem):
    wid = lax.axis_index("s") * _NC + lax.axis_index("c")
    pltpu.sync_copy(champ1_hbm.at[wid], idx1_v)
    pltpu.sync_copy(champ2_hbm.at[wid], idx2_v)

    copies = []
    for j in range(_NCHUNK):
        sl = pl.ds(j * _CHUNK, _CHUNK)
        copies.append(
            pltpu.async_copy(w_hbm.at[idx1_v.at[j]], rows1_v.at[sl], sem))
        copies.append(
            pltpu.async_copy(w_hbm.at[idx2_v.at[j]], rows2_v.at[sl], sem))
    for c in copies:
        c.wait()

    def group_body(g, carry):
        row0 = pl.multiple_of(g * _LANES, _LANES)
        rows = row0 + lax.iota(jnp.int32, _LANES)
        acc = jnp.zeros((_LANES,), jnp.float32)
        for d in range(_NEMB):
            col = jnp.full((_LANES,), d, jnp.int32)
            a = plsc.load_gather(rows1_v, [rows, col])
            b = plsc.load_gather(rows2_v, [rows, col])
            acc = acc + a * b
        out_v[pl.ds(row0, _LANES)] = acc
        return carry

    lax.fori_loop(0, _BPW // _LANES, group_body, 0)
    pltpu.sync_copy(out_v, out_hbm.at[pl.ds(wid * _BPW, _BPW)])


def kernel(champ1, champ2, W):
    c1 = champ1.astype(jnp.int32).reshape(_NW, _NCHUNK, _CHUNK)
    c2 = champ2.astype(jnp.int32).reshape(_NW, _NCHUNK, _CHUNK)
    out = _sc_embed_dot(c1, c2, W)
    return out.reshape(_BATCH, 1, 1)
